# baseline (device time: 21550 ns/iter reference)
import jax
import jax.numpy as jnp
from jax import lax
from jax.experimental import pallas as pl
from jax.experimental.pallas import tpu as pltpu

N_DEV = 8
B = 2
SQ = 256
D = 768
HQ_LOC = 8
HKV_LOC = 2
GROUP = 4
DH = 64
SKV = 512
HC = SQ // N_DEV
N_PEERS = N_DEV - 1
SCALE = 0.125


def kernel(x, Wq, Wo, K_ext, V_ext):
    my_i = lax.axis_index("i")
    k_flat = K_ext.reshape(B, SKV, 16 * DH)
    v_flat = V_ext.reshape(B, SKV, 16 * DH)
    start = my_i * (HKV_LOC * DH)
    k_loc = lax.dynamic_slice(k_flat, (0, 0, start), (B, SKV, HKV_LOC * DH))
    v_loc = lax.dynamic_slice(v_flat, (0, 0, start), (B, SKV, HKV_LOC * DH))
    x2 = x.reshape(B * SQ, D)

    def body(x_ref, wq_ref, wo_ref, k_ref, v_ref, out_ref,
             stage_ref, red_ref, rs_ref, ag_ref,
             rs_send, rs_recv, ag_send, ag_recv):
        i = lax.axis_index("i")

        barrier = pltpu.get_barrier_semaphore()
        for o in range(1, N_DEV):
            pl.semaphore_signal(barrier, inc=1,
                                device_id=(lax.rem(i + o, N_DEV),),
                                device_id_type=pl.DeviceIdType.MESH)

        wq = wq_ref[...].astype(jnp.bfloat16)
        wo = wo_ref[...].astype(jnp.bfloat16)

        def partial_batch(b):
            xb = x_ref[b * SQ:(b + 1) * SQ, :].astype(jnp.bfloat16)
            q = lax.dot_general(xb, wq, (((1,), (0,)), ((), ())),
                                preferred_element_type=jnp.float32)
            q = (q * SCALE).astype(jnp.bfloat16)
            ones = jnp.ones((SKV, 1), jnp.bfloat16)
            cols = [None] * HQ_LOC
            for c in range(HKV_LOC):
                qg = jnp.concatenate(
                    [q[:, h * DH:(h + 1) * DH]
                     for h in range(c * GROUP, (c + 1) * GROUP)], axis=0)
                kc = k_ref[b, :, c * DH:(c + 1) * DH].astype(jnp.bfloat16)
                vc = v_ref[b, :, c * DH:(c + 1) * DH].astype(jnp.bfloat16)
                s = lax.dot_general(qg, kc, (((1,), (1,)), ((), ())),
                                    preferred_element_type=jnp.float32)
                p = jnp.exp(s).astype(jnp.bfloat16)
                l = lax.dot_general(p, ones, (((1,), (0,)), ((), ())),
                                    preferred_element_type=jnp.float32)
                o_ = lax.dot_general(p, vc, (((1,), (0,)), ((), ())),
                                     preferred_element_type=jnp.float32)
                o_ = o_ / l
                for g in range(GROUP):
                    cols[c * GROUP + g] = o_[g * SQ:(g + 1) * SQ, :]
            attn_b = jnp.concatenate(cols, axis=1).astype(jnp.bfloat16)
            return lax.dot_general(attn_b, wo, (((1,), (0,)), ((), ())),
                                   preferred_element_type=jnp.float32)

        for b in range(B):
            part = partial_batch(b)
            for c in range(N_DEV):
                stage_ref[b * N_DEV + c] = part[c * HC:(c + 1) * HC, :].astype(
                    jnp.bfloat16)
            if b == 0:
                pl.semaphore_wait(barrier, N_PEERS)
            for c in range(N_DEV):
                @pl.when(i != c)
                def _():
                    rdma = pltpu.make_async_remote_copy(
                        src_ref=stage_ref.at[b * N_DEV + c],
                        dst_ref=rs_ref.at[b * N_DEV + i],
                        send_sem=rs_send.at[b * N_DEV + c],
                        recv_sem=rs_recv.at[b * N_DEV + i],
                        device_id=(c,),
                        device_id_type=pl.DeviceIdType.MESH,
                    )
                    rdma.start()

        for b in range(B):
            rs_ref[pl.ds(b * N_DEV + i, 1)] = stage_ref[pl.ds(b * N_DEV + i, 1)]
            for j in range(N_DEV):
                @pl.when(i != j)
                def _():
                    rdma = pltpu.make_async_remote_copy(
                        src_ref=stage_ref.at[b * N_DEV + j],
                        dst_ref=rs_ref.at[b * N_DEV + j],
                        send_sem=rs_send.at[b * N_DEV + j],
                        recv_sem=rs_recv.at[b * N_DEV + j],
                        device_id=(j,), device_id_type=pl.DeviceIdType.MESH,
                    )
                    rdma.wait_recv()
            red = rs_ref[b * N_DEV].astype(jnp.float32)
            for j in range(1, N_DEV):
                red = red + rs_ref[b * N_DEV + j].astype(jnp.float32)
            redb = red.astype(jnp.bfloat16)
            red_ref[b] = redb
            for j in range(N_DEV):
                @pl.when(i != j)
                def _():
                    rdma = pltpu.make_async_remote_copy(
                        src_ref=red_ref.at[b],
                        dst_ref=ag_ref.at[b * N_DEV + i],
                        send_sem=ag_send.at[b * N_DEV + j],
                        recv_sem=ag_recv.at[b * N_DEV + i],
                        device_id=(j,),
                        device_id_type=pl.DeviceIdType.MESH,
                    )
                    rdma.start()
            out_ref[pl.ds(b * SQ + i * HC, HC), :] = redb

        for b in range(B):
            for j in range(N_DEV):
                @pl.when(i != j)
                def _():
                    rdma = pltpu.make_async_remote_copy(
                        src_ref=red_ref.at[b],
                        dst_ref=ag_ref.at[b * N_DEV + j],
                        send_sem=ag_send.at[b * N_DEV + j],
                        recv_sem=ag_recv.at[b * N_DEV + j],
                        device_id=(j,), device_id_type=pl.DeviceIdType.MESH,
                    )
                    rdma.wait_recv()
                    out_ref[b * SQ + j * HC:b * SQ + (j + 1) * HC, :] = (
                        ag_ref[b * N_DEV + j])

        for b in range(B):
            for j in range(N_DEV):
                @pl.when(i != j)
                def _():
                    r1 = pltpu.make_async_remote_copy(
                        src_ref=stage_ref.at[b * N_DEV + j],
                        dst_ref=rs_ref.at[b * N_DEV + j],
                        send_sem=rs_send.at[b * N_DEV + j],
                        recv_sem=rs_recv.at[b * N_DEV + j],
                        device_id=(j,), device_id_type=pl.DeviceIdType.MESH,
                    )
                    r1.wait_send()
                    r2 = pltpu.make_async_remote_copy(
                        src_ref=red_ref.at[b],
                        dst_ref=ag_ref.at[b * N_DEV + j],
                        send_sem=ag_send.at[b * N_DEV + j],
                        recv_sem=ag_recv.at[b * N_DEV + j],
                        device_id=(j,), device_id_type=pl.DeviceIdType.MESH,
                    )
                    r2.wait_send()

    out2 = pl.pallas_call(
        body,
        out_shape=jax.ShapeDtypeStruct((B * SQ, D), jnp.bfloat16),
        in_specs=[pl.BlockSpec(memory_space=pltpu.VMEM)] * 5,
        out_specs=pl.BlockSpec(memory_space=pltpu.VMEM),
        scratch_shapes=[
            pltpu.VMEM((B * N_DEV, HC, D), jnp.bfloat16),
            pltpu.VMEM((B, HC, D), jnp.bfloat16),
            pltpu.VMEM((B * N_DEV, HC, D), jnp.bfloat16),
            pltpu.VMEM((B * N_DEV, HC, D), jnp.bfloat16),
            pltpu.SemaphoreType.DMA((B * N_DEV,)),
            pltpu.SemaphoreType.DMA((B * N_DEV,)),
            pltpu.SemaphoreType.DMA((B * N_DEV,)),
            pltpu.SemaphoreType.DMA((B * N_DEV,)),
        ],
        compiler_params=pltpu.CompilerParams(collective_id=0),
    )(x2, Wq, Wo, k_loc, v_loc)
    return out2.reshape(B, SQ, D)


# device time: 21171 ns/iter; 1.0179x vs baseline; 1.0179x over previous
import jax
import jax.numpy as jnp
from jax import lax
from jax.experimental import pallas as pl
from jax.experimental.pallas import tpu as pltpu

N_DEV = 8
B = 2
SQ = 256
D = 768
HQ_LOC = 8
HKV_LOC = 2
GROUP = 4
DH = 64
SKV = 512
HC = SQ // N_DEV
N_PEERS = N_DEV - 1
SCALE = 0.125


def kernel(x, Wq, Wo, K_ext, V_ext):
    my_i = lax.axis_index("i")
    k_flat = K_ext.reshape(B, SKV, 16 * DH)
    v_flat = V_ext.reshape(B, SKV, 16 * DH)
    start = my_i * (HKV_LOC * DH)
    k_loc = lax.dynamic_slice(k_flat, (0, 0, start), (B, SKV, HKV_LOC * DH))
    v_loc = lax.dynamic_slice(v_flat, (0, 0, start), (B, SKV, HKV_LOC * DH))
    x2 = x.reshape(B * SQ, D)

    def body(x_ref, wq_ref, wo_ref, k_ref, v_ref, out_ref,
             stage_ref, red_ref, rs_ref, ag_ref,
             rs_send, rs_recv, ag_send, ag_recv):
        i = lax.axis_index("i")

        barrier = pltpu.get_barrier_semaphore()
        for o in range(1, N_DEV):
            pl.semaphore_signal(barrier, inc=1,
                                device_id=(lax.rem(i + o, N_DEV),),
                                device_id_type=pl.DeviceIdType.MESH)

        wq = wq_ref[...].astype(jnp.bfloat16)
        wo = wo_ref[...].astype(jnp.bfloat16)

        def partial_batch(b):
            xb = x_ref[b * SQ:(b + 1) * SQ, :].astype(jnp.bfloat16)
            q = lax.dot_general(xb, wq, (((1,), (0,)), ((), ())),
                                preferred_element_type=jnp.float32)
            q = q.astype(jnp.bfloat16)
            cols = [None] * HQ_LOC
            for c in range(HKV_LOC):
                qg = jnp.concatenate(
                    [q[:, h * DH:(h + 1) * DH]
                     for h in range(c * GROUP, (c + 1) * GROUP)], axis=0)
                kc = k_ref[b, :, c * DH:(c + 1) * DH].astype(jnp.bfloat16)
                vc = v_ref[b, :, c * DH:(c + 1) * DH].astype(jnp.bfloat16)
                s = lax.dot_general(qg, kc, (((1,), (1,)), ((), ())),
                                    preferred_element_type=jnp.float32)
                s = s * SCALE
                p = jnp.exp(s)
                l = jnp.sum(p, axis=-1, keepdims=True)
                o_ = lax.dot_general(p.astype(jnp.bfloat16), vc,
                                     (((1,), (0,)), ((), ())),
                                     preferred_element_type=jnp.float32)
                o_ = o_ / l
                for g in range(GROUP):
                    cols[c * GROUP + g] = o_[g * SQ:(g + 1) * SQ, :]
            attn_b = jnp.concatenate(cols, axis=1).astype(jnp.bfloat16)
            return lax.dot_general(attn_b, wo, (((1,), (0,)), ((), ())),
                                   preferred_element_type=jnp.float32)

        for b in range(B):
            part = partial_batch(b)
            for c in range(N_DEV):
                stage_ref[b * N_DEV + c] = part[c * HC:(c + 1) * HC, :].astype(
                    jnp.bfloat16)
            if b == 0:
                pl.semaphore_wait(barrier, N_PEERS)
            for c in range(N_DEV):
                @pl.when(i != c)
                def _():
                    rdma = pltpu.make_async_remote_copy(
                        src_ref=stage_ref.at[b * N_DEV + c],
                        dst_ref=rs_ref.at[b * N_DEV + i],
                        send_sem=rs_send.at[b * N_DEV + c],
                        recv_sem=rs_recv.at[b * N_DEV + i],
                        device_id=(c,),
                        device_id_type=pl.DeviceIdType.MESH,
                    )
                    rdma.start()

        for b in range(B):
            rs_ref[pl.ds(b * N_DEV + i, 1)] = stage_ref[pl.ds(b * N_DEV + i, 1)]
            for j in range(N_DEV):
                @pl.when(i != j)
                def _():
                    rdma = pltpu.make_async_remote_copy(
                        src_ref=stage_ref.at[b * N_DEV + j],
                        dst_ref=rs_ref.at[b * N_DEV + j],
                        send_sem=rs_send.at[b * N_DEV + j],
                        recv_sem=rs_recv.at[b * N_DEV + j],
                        device_id=(j,), device_id_type=pl.DeviceIdType.MESH,
                    )
                    rdma.wait_recv()
            red = rs_ref[b * N_DEV].astype(jnp.float32)
            for j in range(1, N_DEV):
                red = red + rs_ref[b * N_DEV + j].astype(jnp.float32)
            redb = red.astype(jnp.bfloat16)
            red_ref[b] = redb
            for j in range(N_DEV):
                @pl.when(i != j)
                def _():
                    rdma = pltpu.make_async_remote_copy(
                        src_ref=red_ref.at[b],
                        dst_ref=ag_ref.at[b * N_DEV + i],
                        send_sem=ag_send.at[b * N_DEV + j],
                        recv_sem=ag_recv.at[b * N_DEV + i],
                        device_id=(j,),
                        device_id_type=pl.DeviceIdType.MESH,
                    )
                    rdma.start()
            out_ref[pl.ds(b * SQ + i * HC, HC), :] = redb

        for b in range(B):
            for j in range(N_DEV):
                @pl.when(i != j)
                def _():
                    rdma = pltpu.make_async_remote_copy(
                        src_ref=red_ref.at[b],
                        dst_ref=ag_ref.at[b * N_DEV + j],
                        send_sem=ag_send.at[b * N_DEV + j],
                        recv_sem=ag_recv.at[b * N_DEV + j],
                        device_id=(j,), device_id_type=pl.DeviceIdType.MESH,
                    )
                    rdma.wait_recv()
                    out_ref[b * SQ + j * HC:b * SQ + (j + 1) * HC, :] = (
                        ag_ref[b * N_DEV + j])

        for b in range(B):
            for j in range(N_DEV):
                @pl.when(i != j)
                def _():
                    r1 = pltpu.make_async_remote_copy(
                        src_ref=stage_ref.at[b * N_DEV + j],
                        dst_ref=rs_ref.at[b * N_DEV + j],
                        send_sem=rs_send.at[b * N_DEV + j],
                        recv_sem=rs_recv.at[b * N_DEV + j],
                        device_id=(j,), device_id_type=pl.DeviceIdType.MESH,
                    )
                    r1.wait_send()
                    r2 = pltpu.make_async_remote_copy(
                        src_ref=red_ref.at[b],
                        dst_ref=ag_ref.at[b * N_DEV + j],
                        send_sem=ag_send.at[b * N_DEV + j],
                        recv_sem=ag_recv.at[b * N_DEV + j],
                        device_id=(j,), device_id_type=pl.DeviceIdType.MESH,
                    )
                    r2.wait_send()

    out2 = pl.pallas_call(
        body,
        out_shape=jax.ShapeDtypeStruct((B * SQ, D), jnp.bfloat16),
        in_specs=[pl.BlockSpec(memory_space=pltpu.VMEM)] * 5,
        out_specs=pl.BlockSpec(memory_space=pltpu.VMEM),
        scratch_shapes=[
            pltpu.VMEM((B * N_DEV, HC, D), jnp.bfloat16),
            pltpu.VMEM((B, HC, D), jnp.bfloat16),
            pltpu.VMEM((B * N_DEV, HC, D), jnp.bfloat16),
            pltpu.VMEM((B * N_DEV, HC, D), jnp.bfloat16),
            pltpu.SemaphoreType.DMA((B * N_DEV,)),
            pltpu.SemaphoreType.DMA((B * N_DEV,)),
            pltpu.SemaphoreType.DMA((B * N_DEV,)),
            pltpu.SemaphoreType.DMA((B * N_DEV,)),
        ],
        compiler_params=pltpu.CompilerParams(collective_id=0),
    )(x2, Wq, Wo, k_loc, v_loc)
    return out2.reshape(B, SQ, D)
